# Initial kernel scaffold; baseline (speedup 1.0000x reference)
import jax
import jax.numpy as jnp
from jax.experimental import pallas as pl


def kernel(features, edge_index, edge_weight, degree):
    raise NotImplementedError("write your pallas kernel here")


# speedup vs baseline: 4.2282x; 4.2282x over previous
import functools

import jax
import jax.numpy as jnp
from jax import lax
from jax.experimental import pallas as pl
from jax.experimental.pallas import tpu as pltpu
from jax.experimental.pallas import tpu_sc as plsc

N_NODES = 10000
N_EDGES = 320000
D_FEAT = 128
LANES = 16

NUM_CORES = 2
NUM_SUBCORES = 16
NUM_WORKERS = NUM_CORES * NUM_SUBCORES
EDGES_PER_WORKER = N_EDGES // NUM_WORKERS
CHUNK = 80
NUM_CHUNKS = EDGES_PER_WORKER // CHUNK
N_PAD = 10240
ROWS_PER_TILE = N_PAD // NUM_SUBCORES
ZROWS = 128

_mesh = plsc.VectorSubcoreMesh(core_axis_name="c", subcore_axis_name="s")


@functools.partial(
    pl.kernel,
    mesh=_mesh,
    out_type=jax.ShapeDtypeStruct((NUM_CORES, N_PAD, D_FEAT), jnp.float32),
    scratch_types=[
        pltpu.VMEM((CHUNK,), jnp.int32),
        pltpu.VMEM((CHUNK,), jnp.int32),
        pltpu.VMEM((CHUNK,), jnp.float32),
        pltpu.VMEM((CHUNK, D_FEAT), jnp.float32),
        pltpu.VMEM((ZROWS, D_FEAT), jnp.float32),
        pltpu.VMEM_SHARED((N_PAD, D_FEAT), jnp.float32),
        pltpu.SemaphoreType.DMA,
    ],
)
def _spmm_partial(x_hbm, col_hbm, row_hbm, w_hbm, out_hbm,
                  colv, rowv, wv, rows, zbuf, acc, sem):
    c = lax.axis_index("c")
    s = lax.axis_index("s")
    wid = s * NUM_CORES + c

    def _zfill(t, carry):
        i = t // (D_FEAT // LANES)
        j = t % (D_FEAT // LANES)
        zbuf[i, pl.ds(j * LANES, LANES)] = jnp.zeros((LANES,), jnp.float32)
        return carry

    lax.fori_loop(0, ZROWS * (D_FEAT // LANES), _zfill, 0)

    def _zacc(i, carry):
        pltpu.sync_copy(zbuf, acc.at[pl.ds(s * ROWS_PER_TILE + i * ZROWS, ZROWS)])
        return carry

    lax.fori_loop(0, ROWS_PER_TILE // ZROWS, _zacc, 0)
    plsc.subcore_barrier()

    base = wid * EDGES_PER_WORKER

    def _chunk(i, carry):
        off = base + i * CHUNK
        pltpu.sync_copy(col_hbm.at[pl.ds(off, CHUNK)], colv)
        pltpu.sync_copy(row_hbm.at[pl.ds(off, CHUNK)], rowv)
        pltpu.sync_copy(w_hbm.at[pl.ds(off, CHUNK)], wv)
        pltpu.async_copy(x_hbm.at[colv], rows, sem).wait()

        gdims = lax.GatherDimensionNumbers(
            offset_dims=(), collapsed_slice_dims=(0,), start_index_map=(0,))

        def _scale(g, carry2):
            wreg = wv[pl.ds(g * LANES, LANES)]
            for e in range(LANES):
                wvec = lax.gather(
                    wreg, jnp.full((LANES, 1), e, jnp.int32), gdims,
                    slice_sizes=(1,),
                    mode=lax.GatherScatterMode.PROMISE_IN_BOUNDS)
                r = g * LANES + e
                for j in range(D_FEAT // LANES):
                    sl = pl.ds(j * LANES, LANES)
                    rows[r, sl] = rows[r, sl] * wvec
            return carry2

        lax.fori_loop(0, CHUNK // LANES, _scale, 0)
        pltpu.sync_copy(rows, acc.at[rowv], add=True)
        return carry

    lax.fori_loop(0, NUM_CHUNKS, _chunk, 0)
    plsc.subcore_barrier()

    pltpu.sync_copy(acc.at[pl.ds(s * ROWS_PER_TILE, ROWS_PER_TILE)],
                    out_hbm.at[c, pl.ds(s * ROWS_PER_TILE, ROWS_PER_TILE)])


_ADD_BS = 512


def _add_body(p_ref, o_ref):
    o_ref[...] = p_ref[0] + p_ref[1]


_combine = pl.pallas_call(
    _add_body,
    grid=(N_PAD // _ADD_BS,),
    in_specs=[pl.BlockSpec((2, _ADD_BS, D_FEAT), lambda i: (0, i, 0))],
    out_specs=pl.BlockSpec((_ADD_BS, D_FEAT), lambda i: (i, 0)),
    out_shape=jax.ShapeDtypeStruct((N_PAD, D_FEAT), jnp.float32),
)


def kernel(features, edge_index, edge_weight, degree):
    row = edge_index[0].astype(jnp.int32)
    col = edge_index[1].astype(jnp.int32)
    w = edge_weight.astype(jnp.float32)
    x0 = jnp.pad(features, ((0, N_PAD - N_NODES), (0, 0)))

    def body(_, x):
        partial = _spmm_partial(x, col, row, w)
        return _combine(partial)

    out = lax.fori_loop(0, degree, body, x0)
    return out[:N_NODES]
